# Initial kernel scaffold; baseline (speedup 1.0000x reference)
import jax
import jax.numpy as jnp
from jax.experimental import pallas as pl


def kernel(x, target):
    raise NotImplementedError("write your pallas kernel here")


# speedup vs baseline: 16.1462x; 16.1462x over previous
import jax
import jax.numpy as jnp
from jax.experimental import pallas as pl
from jax.experimental.pallas import tpu as pltpu

_BINS = 10
_ROWS = 16384
_COLS = 1024
_BLK = 512
_STEPS = _ROWS // _BLK


def _ghm_kernel(x_ref, t_ref, out_ref, cnt_ref, loss_ref):
    i = pl.program_id(0)

    @pl.when(i == 0)
    def _init():
        cnt_ref[...] = jnp.zeros_like(cnt_ref)
        loss_ref[...] = jnp.zeros_like(loss_ref)

    x = x_ref[...]
    t = t_ref[...]
    ax = jnp.abs(x)
    enax = jnp.exp(-ax)
    sg = jnp.where(x >= 0.0, 1.0 / (1.0 + enax), enax / (1.0 + enax))
    g = jnp.abs(sg - t)
    idx = jnp.floor(g * (_BINS - 0.0001)).astype(jnp.int32)
    loss = jnp.maximum(x, 0.0) - x * t + jnp.log1p(enax)
    for b in range(_BINS):
        m = idx == b
        cnt_ref[b, :] += jnp.sum(m.astype(jnp.float32), axis=0)
        loss_ref[b, :] += jnp.sum(jnp.where(m, loss, 0.0), axis=0)

    @pl.when(i == _STEPS - 1)
    def _final():
        cs = [jnp.sum(cnt_ref[b, :]) for b in range(_BINS)]
        ls = [jnp.sum(loss_ref[b, :]) for b in range(_BINS)]
        ne = cs[0] * 0.0
        for c in cs:
            ne = ne + jnp.where(c > 0.0, 1.0, 0.0)
        acc = cs[0] * 0.0
        for c, l in zip(cs, ls):
            gd = jnp.maximum(c * ne, 1e-6)
            acc = acc + (jnp.float32(_ROWS) / gd) * l
        out_ref[0, 0] = acc / jnp.float32(_ROWS * _COLS)


def kernel(x, target):
    out = pl.pallas_call(
        _ghm_kernel,
        grid=(_STEPS,),
        in_specs=[
            pl.BlockSpec((_BLK, _COLS), lambda i: (i, 0)),
            pl.BlockSpec((_BLK, _COLS), lambda i: (i, 0)),
        ],
        out_specs=pl.BlockSpec(
            (1, 1), lambda i: (0, 0), memory_space=pltpu.SMEM
        ),
        out_shape=jax.ShapeDtypeStruct((1, 1), jnp.float32),
        scratch_shapes=[
            pltpu.VMEM((_BINS, _COLS), jnp.float32),
            pltpu.VMEM((_BINS, _COLS), jnp.float32),
        ],
        compiler_params=pltpu.CompilerParams(
            dimension_semantics=("arbitrary",),
        ),
    )(x, target)
    return out[0, 0]


# speedup vs baseline: 26.7904x; 1.6592x over previous
import jax
import jax.numpy as jnp
from jax.experimental import pallas as pl
from jax.experimental.pallas import tpu as pltpu

_BINS = 10
_ROWS = 16384
_COLS = 1024
_BLK = 512
_STEPS = _ROWS // _BLK
_CH = 8
_NCH = _BLK // _CH
_SCALE = float(jnp.float32(_BINS - 0.0001))


def _fold(v):
    acc = v[:, 0:128]
    for k in range(1, _COLS // 128):
        acc = acc + v[:, k * 128:(k + 1) * 128]
    return acc


def _ghm_kernel(x_ref, t_ref, out_ref, accL_ref, accC_ref):
    i = pl.program_id(0)

    def chunk_body(c, carry):
        accL, accC, tot = carry
        x = x_ref[pl.ds(c * _CH, _CH), :]
        t = t_ref[pl.ds(c * _CH, _CH), :]
        enax = jnp.exp(-jnp.abs(x))
        r = 1.0 / (1.0 + enax)
        sg = jnp.where(x >= 0.0, r, 1.0 - r)
        y = jnp.abs(sg - t) * _SCALE
        idx = jnp.floor(y).astype(jnp.int32)
        loss = jnp.maximum(x, 0.0) - x * t + jnp.log1p(enax)
        newL, newC = [], []
        for b in range(1, _BINS):
            m = idx == b
            newL.append(accL[b - 1] + _fold(jnp.where(m, loss, 0.0)))
            newC.append(accC[b - 1] + _fold(jnp.where(m, 1.0, 0.0)))
        return newL, newC, tot + _fold(loss)

    @pl.when(i == 0)
    def _init():
        accL_ref[...] = jnp.zeros_like(accL_ref)
        accC_ref[...] = jnp.zeros_like(accC_ref)

    accL0 = [accL_ref[b] for b in range(_BINS)]
    accC0 = [accC_ref[b] for b in range(_BINS - 1)]
    tot0 = accC_ref[_BINS - 1]
    accL, accC, tot = jax.lax.fori_loop(
        0, _NCH, chunk_body, (accL0[: _BINS - 1], accC0, tot0)
    )
    for b in range(_BINS - 1):
        accL_ref[b] = accL[b]
        accC_ref[b] = accC[b]
    accC_ref[_BINS - 1] = tot

    @pl.when(i == _STEPS - 1)
    def _final():
        cs = [jnp.sum(accC_ref[b]) for b in range(_BINS - 1)]
        ls = [jnp.sum(accL_ref[b]) for b in range(_BINS - 1)]
        ltot = jnp.sum(accC_ref[_BINS - 1])
        c0 = jnp.float32(_ROWS * _COLS)
        l0 = ltot
        for c, l in zip(cs, ls):
            c0 = c0 - c
            l0 = l0 - l
        cs = [c0] + cs
        ls = [l0] + ls
        ne = c0 * 0.0
        for c in cs:
            ne = ne + jnp.where(c > 0.0, 1.0, 0.0)
        acc = c0 * 0.0
        for c, l in zip(cs, ls):
            gd = jnp.maximum(c * ne, 1e-6)
            acc = acc + (jnp.float32(_ROWS) / gd) * l
        out_ref[0, 0] = acc / jnp.float32(_ROWS * _COLS)


def kernel(x, target):
    out = pl.pallas_call(
        _ghm_kernel,
        grid=(_STEPS,),
        in_specs=[
            pl.BlockSpec((_BLK, _COLS), lambda i: (i, 0)),
            pl.BlockSpec((_BLK, _COLS), lambda i: (i, 0)),
        ],
        out_specs=pl.BlockSpec(
            (1, 1), lambda i: (0, 0), memory_space=pltpu.SMEM
        ),
        out_shape=jax.ShapeDtypeStruct((1, 1), jnp.float32),
        scratch_shapes=[
            pltpu.VMEM((_BINS, _CH, 128), jnp.float32),
            pltpu.VMEM((_BINS, _CH, 128), jnp.float32),
        ],
        compiler_params=pltpu.CompilerParams(
            dimension_semantics=("arbitrary",),
        ),
    )(x, target)
    return out[0, 0]


# speedup vs baseline: 29.1674x; 1.0887x over previous
import jax
import jax.numpy as jnp
import numpy as np
from jax.experimental import pallas as pl
from jax.experimental.pallas import tpu as pltpu

_BINS = 10
_ROWS = 16384
_COLS = 1024
_BLK = 512
_STEPS = _ROWS // _BLK
_CH = 16
_NCH = _BLK // _CH
_SCALE = float(np.float32(_BINS - 0.0001))


def _fold(v):
    acc = v[:, 0:128]
    for k in range(1, _COLS // 128):
        acc = acc + v[:, k * 128:(k + 1) * 128]
    while acc.shape[0] > 8:
        h = acc.shape[0] // 2
        acc = acc[0:h, :] + acc[h:, :]
    return acc


def _ghm_kernel(x_ref, t_ref, out_ref, accL_ref, accC_ref):
    i = pl.program_id(0)

    def chunk_body(c, carry):
        accL, accC, tot = carry
        x = x_ref[pl.ds(c * _CH, _CH), :]
        t = t_ref[pl.ds(c * _CH, _CH), :]
        enax = jnp.exp(-jnp.abs(x))
        r = 1.0 / (1.0 + enax)
        sg = jnp.where(x >= 0.0, r, 1.0 - r)
        y = jnp.abs(sg - t) * _SCALE
        idx = jnp.floor(y).astype(jnp.int32)
        loss = jnp.maximum(x, 0.0) - x * t + jnp.log1p(enax)
        newL, newC = [], []
        for b in range(1, _BINS):
            m = idx == b
            newL.append(accL[b - 1] + _fold(jnp.where(m, loss, 0.0)))
            newC.append(accC[b - 1] + _fold(jnp.where(m, 1.0, 0.0)))
        return newL, newC, tot + _fold(loss)

    @pl.when(i == 0)
    def _init():
        accL_ref[...] = jnp.zeros_like(accL_ref)
        accC_ref[...] = jnp.zeros_like(accC_ref)

    accL0 = [accL_ref[b] for b in range(_BINS)]
    accC0 = [accC_ref[b] for b in range(_BINS - 1)]
    tot0 = accC_ref[_BINS - 1]
    accL, accC, tot = jax.lax.fori_loop(
        0, _NCH, chunk_body, (accL0[: _BINS - 1], accC0, tot0)
    )
    for b in range(_BINS - 1):
        accL_ref[b] = accL[b]
        accC_ref[b] = accC[b]
    accC_ref[_BINS - 1] = tot

    @pl.when(i == _STEPS - 1)
    def _final():
        cs = [jnp.sum(accC_ref[b]) for b in range(_BINS - 1)]
        ls = [jnp.sum(accL_ref[b]) for b in range(_BINS - 1)]
        ltot = jnp.sum(accC_ref[_BINS - 1])
        c0 = jnp.float32(_ROWS * _COLS)
        l0 = ltot
        for c, l in zip(cs, ls):
            c0 = c0 - c
            l0 = l0 - l
        cs = [c0] + cs
        ls = [l0] + ls
        ne = c0 * 0.0
        for c in cs:
            ne = ne + jnp.where(c > 0.0, 1.0, 0.0)
        acc = c0 * 0.0
        for c, l in zip(cs, ls):
            gd = jnp.maximum(c * ne, 1e-6)
            acc = acc + (jnp.float32(_ROWS) / gd) * l
        out_ref[0, 0] = acc / jnp.float32(_ROWS * _COLS)


def kernel(x, target):
    out = pl.pallas_call(
        _ghm_kernel,
        grid=(_STEPS,),
        in_specs=[
            pl.BlockSpec((_BLK, _COLS), lambda i: (i, 0)),
            pl.BlockSpec((_BLK, _COLS), lambda i: (i, 0)),
        ],
        out_specs=pl.BlockSpec(
            (1, 1), lambda i: (0, 0), memory_space=pltpu.SMEM
        ),
        out_shape=jax.ShapeDtypeStruct((1, 1), jnp.float32),
        scratch_shapes=[
            pltpu.VMEM((_BINS, 8, 128), jnp.float32),
            pltpu.VMEM((_BINS, 8, 128), jnp.float32),
        ],
        compiler_params=pltpu.CompilerParams(
            dimension_semantics=("arbitrary",),
        ),
    )(x, target)
    return out[0, 0]
